# baseline (speedup 1.0000x reference)
import functools

import jax
import jax.numpy as jnp
from jax import lax
from jax.experimental import pallas as pl
from jax.experimental.pallas import tpu as pltpu
from jax.experimental.pallas import tpu_sc as plsc

N_NODES = 10000
HALF = N_NODES // 2
EMB = 16
OUT = 16
NEF = 4
HID = EMB * OUT
E = 160000

NC = 2
NS = 16
NW = NC * NS

G_PAD = 10240
G_PER_W = G_PAD // NW
G_CHUNKS, G_CW = 5, 64
E_PAD = NW * 40 * 128
E_PER_W = E_PAD // NW
EC, ECW = 40, 128
EBLK = 10240
EGRID = E_PAD // EBLK
NB = EGRID
MSGW = 32
N_ACC = 10240
ROWS_PER_TILE = N_ACC // NS


def _sc_mesh():
    return plsc.VectorSubcoreMesh(
        core_axis_name="c", subcore_axis_name="s",
        num_cores=NC, num_subcores=NS)


def _sc_params():
    return pltpu.CompilerParams(use_tc_tiling_on_sc=False,
                                needs_layout_passes=False)


def _worker_id():
    return lax.axis_index("s") * NC + lax.axis_index("c")


@functools.cache
def _gather_fn(ec):
    e_h = NW * ec * ECW
    e_per_w = ec * ECW
    @functools.partial(
        pl.kernel,
        out_type=jax.ShapeDtypeStruct((e_h, 128), jnp.float32),
        mesh=_sc_mesh(),
        compiler_params=_sc_params(),
        scratch_types=[
            pltpu.VMEM((G_PAD,), jnp.int32),
            pltpu.VMEM((ec, ECW), jnp.int32),
            pltpu.VMEM((ec, ECW), jnp.int32),
            pltpu.VMEM((4, ECW, EMB), jnp.float32),
            pltpu.SemaphoreType.DMA,
            pltpu.SemaphoreType.DMA,
            pltpu.SemaphoreType.DMA,
            pltpu.SemaphoreType.DMA,
            pltpu.SemaphoreType.DMA,
            pltpu.SemaphoreType.DMA,
            pltpu.SemaphoreType.DMA,
            pltpu.SemaphoreType.DMA,
        ],
    )
    def _gather(table_hbm, idxall_hbm, srcb_hbm, xs_hbm,
                idxall_v, src_v, eidx_v, erow_v,
                semg0, semg1, semg2, semg3, semw0, semw1, semw2, semw3):
        wid = _worker_id()
        pltpu.sync_copy(idxall_hbm, idxall_v)
        pltpu.sync_copy(srcb_hbm.at[wid], src_v)
        for j in range(ec):
            for m in range(ECW // 16):
                sv = src_v[j, pl.ds(m * 16, 16)]
                ev = plsc.load_gather(idxall_v, [sv])
                eidx_v[j, pl.ds(m * 16, 16)] = ev
        semg = (semg0, semg1, semg2, semg3)
        semw = (semw0, semw1, semw2, semw3)

        def fire_gather(j, b):
            return pltpu.async_copy(
                table_hbm.at[eidx_v.at[j]], erow_v.at[b], semg[b])

        def fire_write(j, b):
            return pltpu.async_copy(
                erow_v.at[b],
                xs_hbm.at[pl.ds(wid * e_per_w + j * ECW, ECW),
                          pl.ds(0, EMB)],
                semw[b])

        gd = [None] * ec
        wd = [None] * ec
        for j in range(3):
            gd[j] = fire_gather(j, j)
        for j in range(ec):
            b = j % 4
            if j + 3 < ec:
                if j >= 1:
                    wd[j - 1].wait()
                gd[j + 3] = fire_gather(j + 3, (j + 3) % 4)
            gd[j].wait()
            wd[j] = fire_write(j, b)
        for j in range(ec - 4, ec):
            wd[j].wait()

    return _gather


def _edge_body(ea_ref, xs_ref, wtc_ref, btc_ref, wtm_ref, btm_ref,
               w1_ref, b1_ref, w2_ref, b2_ref, out_ref):
    ea = jnp.transpose(ea_ref[...])
    feats = ea[:, 0:NEF]
    m = ea[:, NEF:NEF + 1]
    xs = xs_ref[...][:, :EMB]
    h = jnp.maximum(
        jnp.dot(feats, w1_ref[...], preferred_element_type=jnp.float32)
        + b1_ref[...], 0.0)
    w = jnp.dot(h, w2_ref[...],
                preferred_element_type=jnp.float32) + b2_ref[...]
    xc = jnp.dot(xs, wtc_ref[...],
                 preferred_element_type=jnp.float32) + btc_ref[...]
    xm = jnp.dot(xs, wtm_ref[...],
                 preferred_element_type=jnp.float32) + btm_ref[...]
    xe = xc * m + xm * (1.0 - m)
    ri = lax.broadcasted_iota(jnp.int32, (EMB, HID), 0)
    rj = lax.broadcasted_iota(jnp.int32, (EMB, HID), 1)
    rmat = (rj // OUT == ri).astype(jnp.float32)
    sj = lax.broadcasted_iota(jnp.int32, (HID, OUT), 0)
    so = lax.broadcasted_iota(jnp.int32, (HID, OUT), 1)
    smat = (sj % OUT == so).astype(jnp.float32)
    xrep = jnp.dot(xe, rmat, preferred_element_type=jnp.float32)
    msg = jnp.dot(xrep * w, smat, preferred_element_type=jnp.float32)
    ones = jnp.ones((EBLK, 1), jnp.float32)
    zeros = jnp.zeros((EBLK, MSGW - OUT - 1), jnp.float32)
    full = jnp.concatenate([msg, ones, zeros], axis=1)
    out_ref[...] = jnp.concatenate(
        [full[(EBLK // 4) * u:(EBLK // 4) * (u + 1)] for u in range(4)], axis=1)


def _edge_mlp(eaT, xs128, wtc, btc, wtm, btm, w1, b1, w2, b2):
    n_edges = xs128.shape[0]
    full = lambda s: pl.BlockSpec(s, lambda i: tuple(0 for _ in s))
    return pl.pallas_call(
        _edge_body,
        grid=(n_edges // EBLK,),
        in_specs=[
            pl.BlockSpec((8, EBLK), lambda i: (0, i)),
            pl.BlockSpec((EBLK, 128), lambda i: (i, 0)),
            full((EMB, EMB)), full((1, EMB)),
            full((EMB, EMB)), full((1, EMB)),
            full((NEF, HID)), full((1, HID)),
            full((HID, HID)), full((1, HID)),
        ],
        out_specs=pl.BlockSpec((EBLK // 4, 128), lambda i: (i, 0)),
        out_shape=jax.ShapeDtypeStruct((n_edges // 4, 128), jnp.float32),
        compiler_params=pltpu.CompilerParams(
            dimension_semantics=("arbitrary",)),
    )(eaT, xs128, wtc, btc, wtm, btm, w1, b1, w2, b2)


@functools.cache
def _scatter_fn(ec):
    @functools.partial(
        pl.kernel,
        out_type=(
            jax.ShapeDtypeStruct((NC, N_ACC, 128), jnp.float32),
            jax.ShapeDtypeStruct((G_PAD, 128), jnp.float32),
        ),
        mesh=_sc_mesh(),
        compiler_params=_sc_params(),
        scratch_types=[
            pltpu.VMEM((ec, ECW), jnp.int32),
            pltpu.VMEM((2, ECW, MSGW), jnp.float32),
            pltpu.VMEM_SHARED((N_ACC, MSGW), jnp.float32),
            pltpu.VMEM((G_PAD,), jnp.int32),
            pltpu.VMEM((G_CHUNKS, G_CW, EMB), jnp.float32),
            pltpu.SemaphoreType.DMA,
            pltpu.SemaphoreType.DMA,
            pltpu.SemaphoreType.DMA,
            pltpu.SemaphoreType.DMA,
            pltpu.SemaphoreType.DMA,
        ],
    )
    def _scatter(msg_hbm, dstb_hbm, zeros_hbm, table_hbm, idxall_hbm,
                 out_hbm, g_hbm, dst_v, buf_v, acc, idxall_v, grow_v,
                 semr0, semr1, sems0, sems1, semn):
        cid = lax.axis_index("c")
        sid = lax.axis_index("s")
        wid = sid * NC + cid
        rbase = sid * ROWS_PER_TILE
        pltpu.sync_copy(zeros_hbm.at[pl.ds(rbase, ROWS_PER_TILE)],
                        acc.at[pl.ds(rbase, ROWS_PER_TILE)])
        pltpu.sync_copy(dstb_hbm.at[wid], dst_v)
        pltpu.sync_copy(idxall_hbm, idxall_v)
        nd = []
        for j in range(G_CHUNKS):
            idx = idxall_v.at[pl.ds(wid * G_PER_W + j * G_CW, G_CW)]
            nd.append(pltpu.async_copy(table_hbm.at[idx], grow_v.at[j], semn))
        for j in range(G_CHUNKS):
            nd[j].wait()
        nw = []
        for j in range(G_CHUNKS):
            nw.append(pltpu.async_copy(
                grow_v.at[j],
                g_hbm.at[pl.ds(wid * G_PER_W + j * G_CW, G_CW),
                         pl.ds(0, EMB)], semn))
        for j in range(G_CHUNKS):
            nw[j].wait()
        plsc.subcore_barrier()
        semr = (semr0, semr1)
        sems = (sems0, sems1)

        def fire_read(j, b):
            return pltpu.async_copy(msg_hbm.at[wid, j], buf_v.at[b], semr[b])

        def fire_scatter(j, b):
            return pltpu.async_copy(
                buf_v.at[b], acc.at[dst_v.at[j]], sems[b], add=True)

        rd = [None] * ec
        sd = [None] * ec
        rd[0] = fire_read(0, 0)
        for j in range(ec):
            b = j & 1
            if j + 1 < ec:
                if j >= 1:
                    sd[j - 1].wait()
                rd[j + 1] = fire_read(j + 1, (j + 1) & 1)
            rd[j].wait()
            sd[j] = fire_scatter(j, b)
        sd[ec - 2].wait()
        sd[ec - 1].wait()
        plsc.subcore_barrier()
        pltpu.sync_copy(acc.at[pl.ds(rbase, ROWS_PER_TILE)],
                        out_hbm.at[cid, pl.ds(rbase, ROWS_PER_TILE),
                                   pl.ds(0, MSGW)])

    return _scatter


def _final_body(p_ref, g_ref, wt_ref, bt_ref, root_ref, bc_ref,
                wo_ref, bo_ref, out_ref):
    p = p_ref[...]
    s = p[0][:, :MSGW] + p[1][:, :MSGW]
    agg = s[:, :OUT]
    cnt = jnp.sum(s[:, OUT:], axis=1, keepdims=True)
    mean = agg / jnp.maximum(cnt, 1.0)
    xh = jnp.dot(g_ref[0][:, :EMB], wt_ref[0],
                 preferred_element_type=jnp.float32) + bt_ref[0]
    oc = mean + jnp.dot(xh, root_ref[...],
                        preferred_element_type=jnp.float32) + bc_ref[...]
    x2 = jnp.maximum(oc, 0.0)
    out_ref[...] = jnp.dot(x2, wo_ref[...],
                           preferred_element_type=jnp.float32) + bo_ref[...]


def _finalize(p0, g, wt, bt, root, bias_conv, w_out, b_out):
    nsub = 5
    blk = HALF // nsub
    full = lambda s: pl.BlockSpec(s, lambda h, i: tuple(0 for _ in s))
    return pl.pallas_call(
        _final_body,
        grid=(2, nsub),
        in_specs=[
            pl.BlockSpec((2, blk, 128), lambda h, i: (0, h * nsub + i, 0)),
            pl.BlockSpec((1, blk, 128), lambda h, i: (h, i, 0)),
            pl.BlockSpec((1, EMB, EMB), lambda h, i: (h, 0, 0)),
            pl.BlockSpec((1, 1, EMB), lambda h, i: (h, 0, 0)),
            full((EMB, OUT)), full((1, OUT)), full((OUT, 1)), full((1, 1)),
        ],
        out_specs=pl.BlockSpec((blk, 1), lambda h, i: (h * nsub + i, 0)),
        out_shape=jax.ShapeDtypeStruct((N_NODES, 1), jnp.float32),
    )(p0, g, wt, bt, root, bias_conv.reshape(1, OUT),
      w_out, b_out.reshape(1, 1))


def _perm_slot(a):
    s = a.shape[1:]
    return a.reshape((NB, 4, EBLK // 4) + s).swapaxes(1, 2).reshape(
        (E_PAD,) + s)


def kernel(x_congressperson, x_committee, edge_index, edge_attr,
           emb_cong, emb_comm, Wt_cong, bt_cong, Wt_comm, bt_comm,
           W1, b1, W2, b2, root, bias_conv, W_out, b_out):
    n_cong = emb_cong.shape[0]
    pad = G_PAD // 2 - HALF
    epad = E_PAD - E

    table = jnp.concatenate([emb_cong, emb_comm], axis=0)
    zpad = jnp.zeros((pad,), jnp.int32)
    idx_all = jnp.concatenate(
        [x_congressperson, zpad, x_committee + n_cong, zpad])

    src = edge_index[0]
    dst = edge_index[1]
    src_adj = src + jnp.where(src >= HALF, pad, 0).astype(src.dtype)
    src_pad = jnp.pad(src_adj, (0, epad))
    dst_pad = jnp.pad(dst, (0, epad), constant_values=N_NODES)

    srcb = src_pad.reshape(NW, EC, ECW)
    xs = _gather_fn(EC)(table, idx_all, srcb)

    mask = (src_pad < HALF).astype(jnp.float32)
    ea5 = jnp.concatenate([
        jnp.pad(edge_attr, ((0, epad), (0, 0))),
        mask[:, None],
        jnp.zeros((E_PAD, 8 - NEF - 1), jnp.float32),
    ], axis=1)
    wargs = (Wt_cong, bt_cong.reshape(1, EMB),
             Wt_comm, bt_comm.reshape(1, EMB),
             W1, b1.reshape(1, HID), W2, b2.reshape(1, HID))
    msg = _edge_mlp(ea5.T, xs, *wargs)

    dstb = _perm_slot(dst_pad).reshape(NW, EC, ECW)
    zeros = jnp.zeros((N_ACC, MSGW), jnp.float32)
    p0, g = _scatter_fn(EC)(
        msg.reshape(NW, EC, ECW, MSGW), dstb, zeros, table, idx_all)

    wt = jnp.stack([Wt_cong, Wt_comm]).astype(jnp.float32)
    bt = jnp.stack([bt_cong, bt_comm]).astype(jnp.float32).reshape(2, 1, EMB)
    return _finalize(p0, g.reshape(2, G_PAD // 2, 128),
                     wt, bt, root, bias_conv, W_out, b_out)


# speedup vs baseline: 1.0080x; 1.0080x over previous
import functools

import jax
import jax.numpy as jnp
from jax import lax
from jax.experimental import pallas as pl
from jax.experimental.pallas import tpu as pltpu
from jax.experimental.pallas import tpu_sc as plsc

N_NODES = 10000
HALF = N_NODES // 2
EMB = 16
OUT = 16
NEF = 4
HID = EMB * OUT
E = 160000

NC = 2
NS = 16
NW = NC * NS

G_PAD = 10240
G_PER_W = G_PAD // NW
G_CHUNKS, G_CW = 5, 64
E_PAD = NW * 40 * 128
E_PER_W = E_PAD // NW
EC, ECW = 40, 128
EBLK = 10240
EGRID = E_PAD // EBLK
NB = EGRID
MSGW = 32
N_ACC = 10240
ROWS_PER_TILE = N_ACC // NS


def _sc_mesh():
    return plsc.VectorSubcoreMesh(
        core_axis_name="c", subcore_axis_name="s",
        num_cores=NC, num_subcores=NS)


def _sc_params():
    return pltpu.CompilerParams(use_tc_tiling_on_sc=False,
                                needs_layout_passes=False)


def _worker_id():
    return lax.axis_index("s") * NC + lax.axis_index("c")


@functools.cache
def _gather_fn(ec, with_nodes=False):
    e_h = NW * ec * ECW
    e_per_w = ec * ECW
    node_outs = (jax.ShapeDtypeStruct((G_PAD, 128), jnp.float32),) \
        if with_nodes else ()
    @functools.partial(
        pl.kernel,
        out_type=node_outs + (
            jax.ShapeDtypeStruct((e_h, 128), jnp.float32),
        ),
        mesh=_sc_mesh(),
        compiler_params=_sc_params(),
        scratch_types=[
            pltpu.VMEM((G_PAD,), jnp.int32),
            pltpu.VMEM((ec, ECW), jnp.int32),
            pltpu.VMEM((G_CHUNKS, G_CW, EMB), jnp.float32),
            pltpu.VMEM((ec, ECW), jnp.int32),
            pltpu.VMEM((4, ECW, EMB), jnp.float32),
            pltpu.SemaphoreType.DMA,
            pltpu.SemaphoreType.DMA,
            pltpu.SemaphoreType.DMA,
            pltpu.SemaphoreType.DMA,
            pltpu.SemaphoreType.DMA,
            pltpu.SemaphoreType.DMA,
            pltpu.SemaphoreType.DMA,
            pltpu.SemaphoreType.DMA,
            pltpu.SemaphoreType.DMA,
        ],
    )
    def _gather(table_hbm, idxall_hbm, srcb_hbm, *refs):
        if with_nodes:
            (g_hbm, xs_hbm, idxall_v, src_v, grow_v, eidx_v, erow_v,
             semg0, semg1, semg2, semg3, semw0, semw1, semw2, semw3,
             semn) = refs
        else:
            (xs_hbm, idxall_v, src_v, grow_v, eidx_v, erow_v,
             semg0, semg1, semg2, semg3, semw0, semw1, semw2, semw3,
             semn) = refs
        wid = _worker_id()
        pltpu.sync_copy(idxall_hbm, idxall_v)
        pltpu.sync_copy(srcb_hbm.at[wid], src_v)
        if with_nodes:
            nd = []
            for j in range(G_CHUNKS):
                idx = idxall_v.at[pl.ds(wid * G_PER_W + j * G_CW, G_CW)]
                nd.append(
                    pltpu.async_copy(table_hbm.at[idx], grow_v.at[j], semn))
            for j in range(G_CHUNKS):
                nd[j].wait()
            nw = []
            for j in range(G_CHUNKS):
                nw.append(pltpu.async_copy(
                    grow_v.at[j],
                    g_hbm.at[pl.ds(wid * G_PER_W + j * G_CW, G_CW),
                             pl.ds(0, EMB)], semn))
            for j in range(G_CHUNKS):
                nw[j].wait()
        for j in range(ec):
            for m in range(ECW // 16):
                sv = src_v[j, pl.ds(m * 16, 16)]
                ev = plsc.load_gather(idxall_v, [sv])
                eidx_v[j, pl.ds(m * 16, 16)] = ev
        semg = (semg0, semg1, semg2, semg3)
        semw = (semw0, semw1, semw2, semw3)

        def fire_gather(j, b):
            return pltpu.async_copy(
                table_hbm.at[eidx_v.at[j]], erow_v.at[b], semg[b])

        def fire_write(j, b):
            return pltpu.async_copy(
                erow_v.at[b],
                xs_hbm.at[pl.ds(wid * e_per_w + j * ECW, ECW),
                          pl.ds(0, EMB)],
                semw[b])

        gd = [None] * ec
        wd = [None] * ec
        for j in range(3):
            gd[j] = fire_gather(j, j)
        for j in range(ec):
            b = j % 4
            if j + 3 < ec:
                if j >= 1:
                    wd[j - 1].wait()
                gd[j + 3] = fire_gather(j + 3, (j + 3) % 4)
            gd[j].wait()
            wd[j] = fire_write(j, b)
        for j in range(ec - 4, ec):
            wd[j].wait()

    return _gather


def _edge_body(ea_ref, xs_ref, wtc_ref, btc_ref, wtm_ref, btm_ref,
               w1_ref, b1_ref, w2_ref, b2_ref, out_ref):
    ea = jnp.transpose(ea_ref[...])
    feats = ea[:, 0:NEF]
    m = ea[:, NEF:NEF + 1]
    xs = xs_ref[...][:, :EMB]
    h = jnp.maximum(
        jnp.dot(feats, w1_ref[...], preferred_element_type=jnp.float32)
        + b1_ref[...], 0.0)
    w = jnp.dot(h, w2_ref[...],
                preferred_element_type=jnp.float32) + b2_ref[...]
    xc = jnp.dot(xs, wtc_ref[...],
                 preferred_element_type=jnp.float32) + btc_ref[...]
    xm = jnp.dot(xs, wtm_ref[...],
                 preferred_element_type=jnp.float32) + btm_ref[...]
    xe = xc * m + xm * (1.0 - m)
    ri = lax.broadcasted_iota(jnp.int32, (EMB, HID), 0)
    rj = lax.broadcasted_iota(jnp.int32, (EMB, HID), 1)
    rmat = (rj // OUT == ri).astype(jnp.float32)
    sj = lax.broadcasted_iota(jnp.int32, (HID, OUT), 0)
    so = lax.broadcasted_iota(jnp.int32, (HID, OUT), 1)
    smat = (sj % OUT == so).astype(jnp.float32)
    xrep = jnp.dot(xe, rmat, preferred_element_type=jnp.float32)
    msg = jnp.dot(xrep * w, smat, preferred_element_type=jnp.float32)
    ones = jnp.ones((EBLK, 1), jnp.float32)
    zeros = jnp.zeros((EBLK, MSGW - OUT - 1), jnp.float32)
    full = jnp.concatenate([msg, ones, zeros], axis=1)
    out_ref[...] = jnp.concatenate(
        [full[(EBLK // 4) * u:(EBLK // 4) * (u + 1)] for u in range(4)], axis=1)


def _edge_mlp(eaT, xs128, wtc, btc, wtm, btm, w1, b1, w2, b2):
    n_edges = xs128.shape[0]
    full = lambda s: pl.BlockSpec(s, lambda i: tuple(0 for _ in s))
    return pl.pallas_call(
        _edge_body,
        grid=(n_edges // EBLK,),
        in_specs=[
            pl.BlockSpec((8, EBLK), lambda i: (0, i)),
            pl.BlockSpec((EBLK, 128), lambda i: (i, 0)),
            full((EMB, EMB)), full((1, EMB)),
            full((EMB, EMB)), full((1, EMB)),
            full((NEF, HID)), full((1, HID)),
            full((HID, HID)), full((1, HID)),
        ],
        out_specs=pl.BlockSpec((EBLK // 4, 128), lambda i: (i, 0)),
        out_shape=jax.ShapeDtypeStruct((n_edges // 4, 128), jnp.float32),
        compiler_params=pltpu.CompilerParams(
            dimension_semantics=("arbitrary",)),
    )(eaT, xs128, wtc, btc, wtm, btm, w1, b1, w2, b2)


@functools.cache
def _scatter_fn(ec):
    @functools.partial(
        pl.kernel,
        out_type=jax.ShapeDtypeStruct((NC, N_ACC, 128), jnp.float32),
        mesh=_sc_mesh(),
        compiler_params=_sc_params(),
        scratch_types=[
            pltpu.VMEM((ec, ECW), jnp.int32),
            pltpu.VMEM((2, ECW, MSGW), jnp.float32),
            pltpu.VMEM_SHARED((N_ACC, MSGW), jnp.float32),
            pltpu.SemaphoreType.DMA,
            pltpu.SemaphoreType.DMA,
            pltpu.SemaphoreType.DMA,
            pltpu.SemaphoreType.DMA,
        ],
    )
    def _scatter(msg_hbm, dstb_hbm, zeros_hbm, out_hbm, dst_v, buf_v, acc,
                 semr0, semr1, sems0, sems1):
        cid = lax.axis_index("c")
        sid = lax.axis_index("s")
        wid = sid * NC + cid
        rbase = sid * ROWS_PER_TILE
        pltpu.sync_copy(zeros_hbm.at[pl.ds(rbase, ROWS_PER_TILE)],
                        acc.at[pl.ds(rbase, ROWS_PER_TILE)])
        pltpu.sync_copy(dstb_hbm.at[wid], dst_v)
        plsc.subcore_barrier()
        semr = (semr0, semr1)
        sems = (sems0, sems1)

        def fire_read(j, b):
            return pltpu.async_copy(msg_hbm.at[wid, j], buf_v.at[b], semr[b])

        def fire_scatter(j, b):
            return pltpu.async_copy(
                buf_v.at[b], acc.at[dst_v.at[j]], sems[b], add=True)

        rd = [None] * ec
        sd = [None] * ec
        rd[0] = fire_read(0, 0)
        for j in range(ec):
            b = j & 1
            if j + 1 < ec:
                if j >= 1:
                    sd[j - 1].wait()
                rd[j + 1] = fire_read(j + 1, (j + 1) & 1)
            rd[j].wait()
            sd[j] = fire_scatter(j, b)
        sd[ec - 2].wait()
        sd[ec - 1].wait()
        plsc.subcore_barrier()
        pltpu.sync_copy(acc.at[pl.ds(rbase, ROWS_PER_TILE)],
                        out_hbm.at[cid, pl.ds(rbase, ROWS_PER_TILE),
                                   pl.ds(0, MSGW)])

    return _scatter


def _final_body(p_ref, g_ref, wt_ref, bt_ref, root_ref, bc_ref,
                wo_ref, bo_ref, out_ref):
    p = p_ref[...]
    s = p[0][:, :MSGW] + p[1][:, :MSGW]
    agg = s[:, :OUT]
    cnt = jnp.sum(s[:, OUT:], axis=1, keepdims=True)
    mean = agg / jnp.maximum(cnt, 1.0)
    xh = jnp.dot(g_ref[0][:, :EMB], wt_ref[0],
                 preferred_element_type=jnp.float32) + bt_ref[0]
    oc = mean + jnp.dot(xh, root_ref[...],
                        preferred_element_type=jnp.float32) + bc_ref[...]
    x2 = jnp.maximum(oc, 0.0)
    out_ref[...] = jnp.dot(x2, wo_ref[...],
                           preferred_element_type=jnp.float32) + bo_ref[...]


def _finalize(p0, g, wt, bt, root, bias_conv, w_out, b_out):
    nsub = 5
    blk = HALF // nsub
    full = lambda s: pl.BlockSpec(s, lambda h, i: tuple(0 for _ in s))
    return pl.pallas_call(
        _final_body,
        grid=(2, nsub),
        in_specs=[
            pl.BlockSpec((2, blk, 128), lambda h, i: (0, h * nsub + i, 0)),
            pl.BlockSpec((1, blk, 128), lambda h, i: (h, i, 0)),
            pl.BlockSpec((1, EMB, EMB), lambda h, i: (h, 0, 0)),
            pl.BlockSpec((1, 1, EMB), lambda h, i: (h, 0, 0)),
            full((EMB, OUT)), full((1, OUT)), full((OUT, 1)), full((1, 1)),
        ],
        out_specs=pl.BlockSpec((blk, 1), lambda h, i: (h * nsub + i, 0)),
        out_shape=jax.ShapeDtypeStruct((N_NODES, 1), jnp.float32),
    )(p0, g, wt, bt, root, bias_conv.reshape(1, OUT),
      w_out, b_out.reshape(1, 1))


def _perm_slot(a):
    s = a.shape[1:]
    return a.reshape((NB, 4, EBLK // 4) + s).swapaxes(1, 2).reshape(
        (E_PAD,) + s)


def kernel(x_congressperson, x_committee, edge_index, edge_attr,
           emb_cong, emb_comm, Wt_cong, bt_cong, Wt_comm, bt_comm,
           W1, b1, W2, b2, root, bias_conv, W_out, b_out):
    n_cong = emb_cong.shape[0]
    pad = G_PAD // 2 - HALF
    epad = E_PAD - E

    table = jnp.concatenate([emb_cong, emb_comm], axis=0)
    zpad = jnp.zeros((pad,), jnp.int32)
    idx_all = jnp.concatenate(
        [x_congressperson, zpad, x_committee + n_cong, zpad])

    src = edge_index[0]
    dst = edge_index[1]
    src_adj = src + jnp.where(src >= HALF, pad, 0).astype(src.dtype)
    src_pad = jnp.pad(src_adj, (0, epad))
    dst_pad = jnp.pad(dst, (0, epad), constant_values=N_NODES)

    srcb = src_pad.reshape(NW, EC, ECW)
    g, xs = _gather_fn(EC, True)(table, idx_all, srcb)

    mask = (src_pad < HALF).astype(jnp.float32)
    ea5 = jnp.concatenate([
        jnp.pad(edge_attr, ((0, epad), (0, 0))),
        mask[:, None],
        jnp.zeros((E_PAD, 8 - NEF - 1), jnp.float32),
    ], axis=1)
    wargs = (Wt_cong, bt_cong.reshape(1, EMB),
             Wt_comm, bt_comm.reshape(1, EMB),
             W1, b1.reshape(1, HID), W2, b2.reshape(1, HID))
    msg = _edge_mlp(ea5.T, xs, *wargs)

    dstb = _perm_slot(dst_pad).reshape(NW, EC, ECW)
    zeros = jnp.zeros((N_ACC, MSGW), jnp.float32)
    p0 = _scatter_fn(EC)(msg.reshape(NW, EC, ECW, MSGW), dstb, zeros)

    wt = jnp.stack([Wt_cong, Wt_comm]).astype(jnp.float32)
    bt = jnp.stack([bt_cong, bt_comm]).astype(jnp.float32).reshape(2, 1, EMB)
    return _finalize(p0, g.reshape(2, G_PAD // 2, 128),
                     wt, bt, root, bias_conv, W_out, b_out)
